# baseline (speedup 1.0000x reference)
import functools

import jax
import jax.numpy as jnp
from jax import lax
from jax.experimental import pallas as pl
from jax.experimental.pallas import tpu as pltpu
from jax.experimental.pallas import tpu_sc as plsc

N = 10000
E = 160000
F = 128
H = 2
NS = 16
EW = E // NS
LANES = 16
BLK = 2000
NBLK = N // BLK
ROWP = 2048
MBLK = 2 * NS * ROWP


def _sc_mask_body(dst_hbm, out_hbm, idx_v, mask_v, sem):
    c = lax.axis_index("c")
    s = lax.axis_index("s")

    cp = pltpu.async_copy(
        dst_hbm.at[pl.ds(c * E + s * EW, EW)], idx_v, sem)

    zeros = jnp.zeros((LANES,), jnp.float32)

    @plsc.parallel_loop(0, N // LANES, unroll=16)
    def _zero(i):
        mask_v[pl.ds(i * LANES, LANES)] = zeros

    cp.wait()

    ones = jnp.ones((LANES,), jnp.float32)

    @plsc.parallel_loop(0, EW // LANES, unroll=16)
    def _scatter(j):
        iv = idx_v[pl.ds(j * LANES, LANES)]
        plsc.store_scatter(mask_v, [iv], ones)

    outs = [
        pltpu.async_copy(
            mask_v.at[pl.ds(i * BLK, BLK)],
            out_hbm.at[pl.ds(i * MBLK + (c * NS + s) * ROWP, BLK)],
            sem)
        for i in range(NBLK)
    ]
    for cp_out in outs:
        cp_out.wait()


_sc_masks = functools.partial(
    pl.kernel,
    mesh=plsc.VectorSubcoreMesh(core_axis_name="c", subcore_axis_name="s"),
    out_type=jax.ShapeDtypeStruct((NBLK * MBLK,), jnp.float32),
    scratch_types=[
        pltpu.VMEM((EW,), jnp.int32),
        pltpu.VMEM((N,), jnp.float32),
        pltpu.SemaphoreType.DMA,
    ],
    compiler_params=pltpu.CompilerParams(
        needs_layout_passes=False, use_tc_tiling_on_sc=False),
)(_sc_mask_body)


def _tc_apply_body(m_ref, hp_ref, hs_ref, bp_ref, bs_ref, op_ref, os_ref):
    ones_row = jnp.ones((1, F), jnp.float32)
    dn = (((0,), (0,)), ((), ()))

    def counts(etype):
        cnt = m_ref[pl.ds(etype * NS * ROWP, BLK)]
        for j in range(1, NS):
            cnt = cnt + m_ref[pl.ds((etype * NS + j) * ROWP, BLK)]
        return lax.dot_general(cnt[None, :], ones_row, dn,
                               preferred_element_type=jnp.float32)

    def bias_row(b_ref):
        return ((b_ref[pl.ds(0, F)] + b_ref[pl.ds(F, F)]) * 0.5)[None, :]

    csp = counts(0)
    cps = counts(1)
    op_ref[...] = jnp.where(csp > 0.0, hp_ref[...], 0.0) + bias_row(bp_ref)
    os_ref[...] = jnp.where(cps > 0.0, hs_ref[...], 0.0) + bias_row(bs_ref)


def kernel(h_sent, h_pair, rel_ctx_sp, rel_ctx_ps, W_src, W_dst, attn_l_sp,
           attn_r_sp, attn_l_ps, attn_r_ps, bias_sent, bias_pair,
           edge_index_sp, edge_index_ps):
    dst_all = jnp.concatenate(
        [edge_index_sp[1], edge_index_ps[1]]).astype(jnp.int32)

    masks = _sc_masks(dst_all)

    out_pair, out_sent = pl.pallas_call(
        _tc_apply_body,
        grid=(NBLK,),
        in_specs=[
            pl.BlockSpec((MBLK,), lambda i: (i,)),
            pl.BlockSpec((BLK, F), lambda i: (i, 0)),
            pl.BlockSpec((BLK, F), lambda i: (i, 0)),
            pl.BlockSpec((H * F,), lambda i: (0,)),
            pl.BlockSpec((H * F,), lambda i: (0,)),
        ],
        out_specs=[
            pl.BlockSpec((BLK, F), lambda i: (i, 0)),
            pl.BlockSpec((BLK, F), lambda i: (i, 0)),
        ],
        out_shape=[
            jax.ShapeDtypeStruct((N, F), jnp.float32),
            jax.ShapeDtypeStruct((N, F), jnp.float32),
        ],
    )(masks, h_pair, h_sent, bias_pair, bias_sent)

    return (out_pair, out_sent)


# speedup vs baseline: 1.2590x; 1.2590x over previous
import functools

import jax
import jax.numpy as jnp
from jax import lax
from jax.experimental import pallas as pl
from jax.experimental.pallas import tpu as pltpu
from jax.experimental.pallas import tpu_sc as plsc

N = 10000
E = 160000
F = 128
H = 2
NS = 16
LANES = 16
BLK = 5000
NBLK = N // BLK
ROWP = 5120
MBLK = 2 * NS * ROWP

ER = E // F
ERP = 1280
WR = ERP // NS
NPAD = 10240


def _tc_extract_body(a_ref, b_ref, o_ref):
    ra = a_ref[1:2, :].reshape(ER, F)
    rb = b_ref[1:2, :].reshape(ER, F)
    pad = jnp.full((ERP - ER, F), N, jnp.int32)
    o_ref[...] = jnp.concatenate([ra, pad, rb, pad], axis=0)


_tc_extract = pl.pallas_call(
    _tc_extract_body,
    out_shape=jax.ShapeDtypeStruct((2 * ERP, F), jnp.int32),
)


def _sc_mask_body(dst_hbm, out_hbm, idx_v, mask_v, sem):
    c = lax.axis_index("c")
    s = lax.axis_index("s")

    cp = pltpu.async_copy(
        dst_hbm.at[pl.ds(c * ERP + s * WR, WR), :], idx_v, sem)

    zeros = jnp.zeros((LANES,), jnp.float32)

    @plsc.parallel_loop(0, NPAD // LANES, unroll=16)
    def _zero(i):
        mask_v[pl.ds(i * LANES, LANES)] = zeros

    cp.wait()

    ones = jnp.ones((LANES,), jnp.float32)

    @plsc.parallel_loop(0, WR, unroll=2)
    def _scatter(r):
        for k in range(F // LANES):
            iv = idx_v[r, pl.ds(k * LANES, LANES)]
            plsc.store_scatter(mask_v, [iv], ones)

    outs = [
        pltpu.async_copy(
            mask_v.at[pl.ds(i * BLK, BLK)],
            out_hbm.at[pl.ds(i * MBLK + (c * NS + s) * ROWP, BLK)],
            sem)
        for i in range(NBLK)
    ]
    for cp_out in outs:
        cp_out.wait()


_sc_masks = functools.partial(
    pl.kernel,
    mesh=plsc.VectorSubcoreMesh(core_axis_name="c", subcore_axis_name="s"),
    out_type=jax.ShapeDtypeStruct((NBLK * MBLK,), jnp.float32),
    scratch_types=[
        pltpu.VMEM((WR, F), jnp.int32),
        pltpu.VMEM((NPAD,), jnp.float32),
        pltpu.SemaphoreType.DMA,
    ],
    compiler_params=pltpu.CompilerParams(
        needs_layout_passes=False, use_tc_tiling_on_sc=False),
)(_sc_mask_body)


def _tc_apply_body(m_ref, hp_ref, hs_ref, bp_ref, bs_ref, op_ref, os_ref):
    ones_row = jnp.ones((1, F), jnp.float32)
    dn = (((0,), (0,)), ((), ()))

    def counts(etype):
        cnt = m_ref[pl.ds(etype * NS * ROWP, BLK)]
        for j in range(1, NS):
            cnt = cnt + m_ref[pl.ds((etype * NS + j) * ROWP, BLK)]
        return lax.dot_general(cnt[None, :], ones_row, dn,
                               preferred_element_type=jnp.float32)

    def bias_row(b_ref):
        return ((b_ref[pl.ds(0, F)] + b_ref[pl.ds(F, F)]) * 0.5)[None, :]

    csp = counts(0)
    cps = counts(1)
    op_ref[...] = jnp.where(csp > 0.0, hp_ref[...], 0.0) + bias_row(bp_ref)
    os_ref[...] = jnp.where(cps > 0.0, hs_ref[...], 0.0) + bias_row(bs_ref)


def kernel(h_sent, h_pair, rel_ctx_sp, rel_ctx_ps, W_src, W_dst, attn_l_sp,
           attn_r_sp, attn_l_ps, attn_r_ps, bias_sent, bias_pair,
           edge_index_sp, edge_index_ps):
    dst_all = _tc_extract(edge_index_sp.astype(jnp.int32),
                          edge_index_ps.astype(jnp.int32))

    masks = _sc_masks(dst_all)

    out_pair, out_sent = pl.pallas_call(
        _tc_apply_body,
        grid=(NBLK,),
        in_specs=[
            pl.BlockSpec((MBLK,), lambda i: (i,)),
            pl.BlockSpec((BLK, F), lambda i: (i, 0)),
            pl.BlockSpec((BLK, F), lambda i: (i, 0)),
            pl.BlockSpec((H * F,), lambda i: (0,)),
            pl.BlockSpec((H * F,), lambda i: (0,)),
        ],
        out_specs=[
            pl.BlockSpec((BLK, F), lambda i: (i, 0)),
            pl.BlockSpec((BLK, F), lambda i: (i, 0)),
        ],
        out_shape=[
            jax.ShapeDtypeStruct((N, F), jnp.float32),
            jax.ShapeDtypeStruct((N, F), jnp.float32),
        ],
    )(masks, h_pair, h_sent, bias_pair, bias_sent)

    return (out_pair, out_sent)
